# baseline (speedup 1.0000x reference)
import functools

import jax
import jax.numpy as jnp
from jax import lax
from jax.experimental import pallas as pl
from jax.experimental.pallas import tpu as pltpu
from jax.experimental.pallas import tpu_sc as plsc


def _sc_fused_gather(X_domains, aT, tabT):
    info = plsc.get_sparse_core_info()
    nc, ns = info.num_cores, info.num_subcores
    nw = nc * ns
    arity, B = aT.shape
    D = tabT.shape[0]
    bpw = B // nw
    mesh = plsc.VectorSubcoreMesh(core_axis_name="c", subcore_axis_name="s",
                                  num_cores=nc)

    @functools.partial(
        pl.kernel,
        out_type=(jax.ShapeDtypeStruct((D, B), jnp.float32),
                  jax.ShapeDtypeStruct((D, B), jnp.float32)),
        mesh=mesh,
        scratch_types=[
            pltpu.VMEM((arity, bpw), jnp.int32),
            pltpu.VMEM((bpw,), jnp.int32),
            pltpu.VMEM((bpw,), jnp.int32),
            pltpu.VMEM((D, bpw), jnp.float32),
            pltpu.VMEM((D, bpw), jnp.float32),
            pltpu.SemaphoreType.DMA,
            pltpu.SemaphoreType.DMA,
        ],
        compiler_params=pltpu.CompilerParams(use_tc_tiling_on_sc=False),
    )
    def gather_kernel(xdom, a_hbm, tab, ehT_out, etT_out,
                      a2_v, ih_v, it_v, ehT_v, etT_v, sem_h, sem_t):
        wid = lax.axis_index("s") * nc + lax.axis_index("c")
        base = wid * bpw
        pltpu.sync_copy(a_hbm.at[:, pl.ds(base, bpw)], a2_v)
        ch = pltpu.async_copy(xdom.at[a2_v.at[0]], ih_v, sem_h)
        ct = pltpu.async_copy(xdom.at[a2_v.at[1]], it_v, sem_t)
        ch.wait()
        ct.wait()
        copies = []
        for f in range(D):
            copies.append(pltpu.async_copy(
                tab.at[f].at[ih_v], ehT_v.at[f], sem_h))
            copies.append(pltpu.async_copy(
                tab.at[f].at[it_v], etT_v.at[f], sem_t))
        for c in copies:
            c.wait()
        pltpu.sync_copy(ehT_v, ehT_out.at[:, pl.ds(base, bpw)])
        pltpu.sync_copy(etT_v, etT_out.at[:, pl.ds(base, bpw)])

    return gather_kernel(X_domains, aT, tabT)


def _mm_body(ehT_ref, etT_ref, w_ref, b_ref, o_ref):
    ehT = ehT_ref[...]
    etT = etT_ref[...]
    D = ehT.shape[0]
    dn = (((0,), (0,)), ((), ()))
    hp = jax.lax.Precision.HIGHEST
    acc = lax.dot_general(ehT, w_ref[0:D, :], dn, precision=hp,
                          preferred_element_type=jnp.float32)
    acc = acc + lax.dot_general(etT, w_ref[D:2 * D, :], dn, precision=hp,
                                preferred_element_type=jnp.float32)
    acc = acc + lax.dot_general(ehT * etT, w_ref[2 * D:3 * D, :], dn,
                                precision=hp,
                                preferred_element_type=jnp.float32)
    o_ref[...] = jnp.tanh(acc + b_ref[...])


def _tc_embed(ehT, etT, W, b):
    D, B = ehT.shape
    K, A = W.shape
    blk = 2048
    return pl.pallas_call(
        _mm_body,
        grid=(B // blk,),
        in_specs=[
            pl.BlockSpec((D, blk), lambda i: (0, i)),
            pl.BlockSpec((D, blk), lambda i: (0, i)),
            pl.BlockSpec((K, A), lambda i: (0, 0)),
            pl.BlockSpec((A,), lambda i: (0,)),
        ],
        out_specs=pl.BlockSpec((blk, A), lambda i: (i, 0)),
        out_shape=jax.ShapeDtypeStruct((B, A), jnp.float32),
    )(ehT, etT, W, b)


def kernel(X_domains, A_predicates, table, W, b):
    aT = A_predicates.T
    tabT = table.T
    ehT, etT = _sc_fused_gather(X_domains, aT, tabT)
    return _tc_embed(ehT, etT, W, b)
